# baseline (speedup 1.0000x reference)
import functools

import jax
import jax.numpy as jnp
from jax import lax
from jax.experimental import pallas as pl
from jax.experimental.pallas import tpu as pltpu
from jax.experimental.pallas import tpu_sc as plsc

_VOCAB = 32000
_D = 2048
_RANK = 16
_SCALING = 2.0

_BTOK = 4 * 4096
_NC, _NS = 2, 16
_NW = _NC * _NS
_TPW = _BTOK // _NW
_CHUNK = 16
_NCHUNK = _TPW // _CHUNK


def _sc_gather(x2d, embed_w, lora_a):
    mesh = plsc.VectorSubcoreMesh(core_axis_name="c", subcore_axis_name="s")

    @functools.partial(
        pl.kernel,
        mesh=mesh,
        out_type=(
            jax.ShapeDtypeStruct((_BTOK, _D), jnp.float32),
            jax.ShapeDtypeStruct((_BTOK, 128), jnp.float32),
        ),
        scratch_types=[
            pltpu.VMEM((_NCHUNK, _CHUNK), jnp.int32),
            pltpu.VMEM((_CHUNK, _D), jnp.float32),
            pltpu.VMEM((_CHUNK, 128), jnp.float32),
            pltpu.SemaphoreType.DMA,
            pltpu.SemaphoreType.DMA,
        ],
    )
    def k(x_hbm, table_hbm, a_hbm, out_hbm, arows_hbm, idx_v, rows_v, av_v,
          gsem, asem):
        wid = lax.axis_index("s") * _NC + lax.axis_index("c")
        tok_base = wid * _TPW
        pltpu.sync_copy(x_hbm.at[pl.ds(wid * _NCHUNK, _NCHUNK)], idx_v)

        def body(j, carry):
            idx_row = idx_v.at[j]
            pltpu.async_copy(table_hbm.at[idx_row], rows_v, gsem).wait()
            pltpu.async_copy(a_hbm.at[idx_row], av_v, asem).wait()
            pltpu.sync_copy(rows_v, out_hbm.at[pl.ds(tok_base + j * _CHUNK, _CHUNK)])
            pltpu.sync_copy(
                av_v, arows_hbm.at[pl.ds(tok_base + j * _CHUNK, _CHUNK)])
            return carry

        lax.fori_loop(0, _NCHUNK, body, 0)

    return k(x2d, embed_w, lora_a)


_BT = 512


def _tc_body(g_ref, a_ref, b_ref, o_ref):
    o_ref[...] = g_ref[...] + jnp.dot(
        a_ref[:, :_RANK], b_ref[...],
        preferred_element_type=jnp.float32) * _SCALING


def _tc_fused(gathered, arows, lora_b):
    return pl.pallas_call(
        _tc_body,
        grid=(_BTOK // _BT,),
        in_specs=[
            pl.BlockSpec((_BT, _D), lambda i: (i, 0)),
            pl.BlockSpec((_BT, 128), lambda i: (i, 0)),
            pl.BlockSpec((_RANK, _D), lambda i: (0, 0)),
        ],
        out_specs=pl.BlockSpec((_BT, _D), lambda i: (i, 0)),
        out_shape=jax.ShapeDtypeStruct((_BTOK, _D), jnp.float32),
    )(gathered, arows, lora_b)


def kernel(x, embed_w, lora_A, lora_B):
    b, s = x.shape
    x2d = x.reshape(_BTOK // _CHUNK, _CHUNK).astype(jnp.int32)
    lora_a_pad = jnp.pad(lora_A, ((0, 0), (0, 128 - _RANK)))
    gathered, arows = _sc_gather(x2d, embed_w, lora_a_pad)
    out = _tc_fused(gathered, arows, lora_B)
    return out.reshape(b, s, _D)


# speedup vs baseline: 1.1609x; 1.1609x over previous
import functools

import jax
import jax.numpy as jnp
from jax import lax
from jax.experimental import pallas as pl
from jax.experimental.pallas import tpu as pltpu
from jax.experimental.pallas import tpu_sc as plsc

_VOCAB = 32000
_D = 2048
_RANK = 16
_SCALING = 2.0

_BTOK = 4 * 4096
_NC, _NS = 2, 16
_NW = _NC * _NS
_TPW = _BTOK // _NW

_CHUNK = 8
_NBUF = 4
_NCHUNK = _TPW // _CHUNK
_NSUP = _NCHUNK // _NBUF

_ACHUNK = 32
_ANBUF = 2
_ANCHUNK = _TPW // _ACHUNK


def _sc_gather(x8, x32, embed_w, lora_a_pad):
    mesh = plsc.VectorSubcoreMesh(core_axis_name="c", subcore_axis_name="s")

    @functools.partial(
        pl.kernel,
        mesh=mesh,
        out_type=(
            jax.ShapeDtypeStruct((_BTOK, _D), jnp.float32),
            jax.ShapeDtypeStruct((_BTOK, 128), jnp.float32),
        ),
        scratch_types=[
            pltpu.VMEM((_NCHUNK, _CHUNK), jnp.int32),
            pltpu.VMEM((_ANCHUNK, _ACHUNK), jnp.int32),
            pltpu.VMEM((_NBUF, _CHUNK, _D), jnp.float32),
            pltpu.VMEM((_ANBUF, _ACHUNK, 128), jnp.float32),
            pltpu.SemaphoreType.DMA((_NBUF,)),
            pltpu.SemaphoreType.DMA((_NBUF,)),
            pltpu.SemaphoreType.DMA((_ANBUF,)),
            pltpu.SemaphoreType.DMA((_ANBUF,)),
        ],
    )
    def k(x8_hbm, x32_hbm, table_hbm, a_hbm, out_hbm, arows_hbm,
          idx_v, idxa_v, rows_v, av_v, gsem, osem, agsem, aosem):
        wid = lax.axis_index("s") * _NC + lax.axis_index("c")
        tok_base = wid * _TPW
        pltpu.sync_copy(x8_hbm.at[pl.ds(wid * _NCHUNK, _NCHUNK)], idx_v)
        pltpu.sync_copy(x32_hbm.at[pl.ds(wid * _ANCHUNK, _ANCHUNK)], idxa_v)

        def fire_g(j, b):
            pltpu.async_copy(table_hbm.at[idx_v.at[j]], rows_v.at[b],
                             gsem.at[b])

        def fire_o(j, b):
            pltpu.async_copy(
                rows_v.at[b],
                out_hbm.at[pl.ds(tok_base + j * _CHUNK, _CHUNK)],
                osem.at[b])

        def wait_g(b):
            pltpu.make_async_copy(table_hbm.at[idx_v.at[0]], rows_v.at[b],
                                  gsem.at[b]).wait()

        def wait_o(b):
            pltpu.make_async_copy(
                rows_v.at[b], out_hbm.at[pl.ds(0, _CHUNK)],
                osem.at[b]).wait()

        def fire_ag(i, ab):
            pltpu.async_copy(a_hbm.at[idxa_v.at[i]], av_v.at[ab],
                             agsem.at[ab])

        def fire_ao(i, ab):
            pltpu.async_copy(
                av_v.at[ab],
                arows_hbm.at[pl.ds(tok_base + i * _ACHUNK, _ACHUNK)],
                aosem.at[ab])

        def wait_ag(ab):
            pltpu.make_async_copy(a_hbm.at[idxa_v.at[0]], av_v.at[ab],
                                  agsem.at[ab]).wait()

        def wait_ao(ab):
            pltpu.make_async_copy(
                av_v.at[ab], arows_hbm.at[pl.ds(0, _ACHUNK)],
                aosem.at[ab]).wait()

        for b in range(_NBUF):
            fire_g(b, b)
        for ab in range(_ANBUF):
            fire_ag(ab, ab)

        def body(i, carry):
            ab = lax.rem(i, _ANBUF)
            for b in range(_NBUF):
                wait_g(b)
                fire_o(i * _NBUF + b, b)
            wait_ag(ab)
            fire_ao(i, ab)
            @pl.when(i < _NSUP - 1)
            def _():
                for b in range(_NBUF):
                    wait_o(b)
                    fire_g((i + 1) * _NBUF + b, b)

            @pl.when(i < _NSUP - _ANBUF)
            def _():
                wait_ao(ab)
                fire_ag(i + _ANBUF, ab)
            return carry

        lax.fori_loop(0, _NSUP, body, 0)
        for b in range(_NBUF):
            wait_o(b)
        for ab in range(_ANBUF):
            wait_ao(ab)

    return k(x8, x32, embed_w, lora_a_pad)


_BT = 512


def _tc_body(g_ref, a_ref, b_ref, o_ref):
    o_ref[...] = g_ref[...] + jnp.dot(
        a_ref[:, :_RANK], b_ref[...],
        preferred_element_type=jnp.float32) * _SCALING


def _tc_fused(gathered, arows, lora_b):
    return pl.pallas_call(
        _tc_body,
        grid=(_BTOK // _BT,),
        in_specs=[
            pl.BlockSpec((_BT, _D), lambda i: (i, 0)),
            pl.BlockSpec((_BT, 128), lambda i: (i, 0)),
            pl.BlockSpec((_RANK, _D), lambda i: (0, 0)),
        ],
        out_specs=pl.BlockSpec((_BT, _D), lambda i: (i, 0)),
        out_shape=jax.ShapeDtypeStruct((_BTOK, _D), jnp.float32),
    )(gathered, arows, lora_b)


def kernel(x, embed_w, lora_A, lora_B):
    b, s = x.shape
    xf = x.reshape(-1).astype(jnp.int32)
    x8 = xf.reshape(_BTOK // _CHUNK, _CHUNK)
    x32 = xf.reshape(_BTOK // _ACHUNK, _ACHUNK)
    lora_a_pad = jnp.pad(lora_A, ((0, 0), (0, 128 - _RANK)))
    gathered, arows = _sc_gather(x8, x32, embed_w, lora_a_pad)
    out = _tc_fused(gathered, arows, lora_B)
    return out.reshape(b, s, _D)
